# Initial kernel scaffold; baseline (speedup 1.0000x reference)
import jax
import jax.numpy as jnp
from jax.experimental import pallas as pl


def kernel(x, edge_index, edge_weight, gcn_w, gcn_b, w_z, b_z, w_r, b_r, w_h, b_h, lin_w, lin_b):
    raise NotImplementedError("write your pallas kernel here")


# speedup vs baseline: 9.3471x; 9.3471x over previous
import functools

import jax
import jax.numpy as jnp
from jax import lax
from jax.experimental import pallas as pl
from jax.experimental.pallas import tpu as pltpu
from jax.experimental.pallas import tpu_sc as plsc

_N = 10000
_E = 320000
_F = 128
_C = 10

_NSUB = 16
_NCORE = 2
_NW = _NSUB * _NCORE
_NPAD = 10240
_STRIPE = _NPAD // _NSUB
_EPW = 10240
_EPAD = _NW * _EPW
_SUB = 128

_sc_mesh = plsc.VectorSubcoreMesh(core_axis_name="c", subcore_axis_name="s")


def _deg_body(dst_hbm, ew_hbm, out_hbm, zb, ewv, dstv, acc, sem):
    c = lax.axis_index("c")
    s = lax.axis_index("s")
    wid = s * _NCORE + c
    base = wid * _EPW

    def zstore(i, _):
        zb[pl.ds(i * 16, 16)] = jnp.zeros((16,), jnp.float32)
        return 0

    lax.fori_loop(0, _STRIPE // 16, zstore, 0)
    pltpu.sync_copy(zb, acc.at[pl.ds(s * _STRIPE, _STRIPE)])
    plsc.subcore_barrier()

    def chunk(k, _):
        cb = base + k * 1024
        pltpu.sync_copy(ew_hbm.at[pl.ds(cb, 1024)], ewv)
        for j in range(8):
            pltpu.sync_copy(dst_hbm.at[pl.ds(cb + j * _SUB, _SUB)], dstv.at[j])
        for j in range(8):
            pltpu.sync_copy(ewv.at[pl.ds(j * _SUB, _SUB)],
                            acc.at[dstv.at[j]], add=True)
        return 0

    lax.fori_loop(0, _EPW // 1024, chunk, 0)
    plsc.subcore_barrier()
    pltpu.sync_copy(acc.at[pl.ds(s * _STRIPE, _STRIPE)],
                    out_hbm.at[c, pl.ds(s * _STRIPE, _STRIPE)])


_deg_call = pl.kernel(
    _deg_body,
    out_type=jax.ShapeDtypeStruct((_NCORE, _NPAD), jnp.float32),
    mesh=_sc_mesh,
    scratch_types=[
        pltpu.VMEM((_STRIPE,), jnp.float32),
        pltpu.VMEM((1024,), jnp.float32),
        pltpu.VMEM((8, _SUB), jnp.int32),
        pltpu.VMEM_SHARED((_NPAD,), jnp.float32),
        pltpu.SemaphoreType.DMA,
    ],
)


_CHUNK = 256
_ZR = 32


def _agg_body(src_hbm, dst_hbm, ew_hbm, xw2_hbm, out_hbm,
              zrow, srcv, dstv, ewv, rows, acc, sem):
    c = lax.axis_index("c")
    s = lax.axis_index("s")
    wid = s * _NCORE + c
    base = wid * _EPW

    def zstore(i, _):
        r = i // 8
        j = i % 8
        zrow[r, pl.ds(j * 16, 16)] = jnp.zeros((16,), jnp.float32)
        return 0

    lax.fori_loop(0, _ZR * 8, zstore, 0)
    for t in range(_STRIPE // _ZR):
        pltpu.sync_copy(zrow, acc.at[pl.ds(s * _STRIPE + t * _ZR, _ZR)])
    plsc.subcore_barrier()

    def chunk(k, _):
        cb = base + k * _CHUNK
        pltpu.sync_copy(ew_hbm.at[pl.ds(cb, _CHUNK)], ewv)
        for j in range(_CHUNK // _SUB):
            pltpu.sync_copy(src_hbm.at[pl.ds(cb + j * _SUB, _SUB)], srcv.at[j])
            pltpu.sync_copy(dst_hbm.at[pl.ds(cb + j * _SUB, _SUB)], dstv.at[j])
        for j in range(_CHUNK // _SUB):
            pltpu.async_copy(xw2_hbm.at[srcv.at[j]],
                             rows.at[pl.ds(j * _SUB, _SUB)], sem).wait()

        def scale(g, _):
            wv = ewv[pl.ds(g * 16, 16)]
            for i in range(16):
                w = wv[i]
                e = g * 16 + i
                for j in range(_F // 16):
                    rows[e, pl.ds(j * 16, 16)] = rows[e, pl.ds(j * 16, 16)] * w
            return 0

        lax.fori_loop(0, _CHUNK // 16, scale, 0)
        for j in range(_CHUNK // _SUB):
            pltpu.sync_copy(rows.at[pl.ds(j * _SUB, _SUB)],
                            acc.at[dstv.at[j]], add=True)
        return 0

    lax.fori_loop(0, _EPW // _CHUNK, chunk, 0)
    plsc.subcore_barrier()
    pltpu.sync_copy(acc.at[pl.ds(s * _STRIPE, _STRIPE)],
                    out_hbm.at[c, pl.ds(s * _STRIPE, _STRIPE)])


_agg_call = pl.kernel(
    _agg_body,
    out_type=jax.ShapeDtypeStruct((_NCORE, _NPAD, _F), jnp.float32),
    mesh=_sc_mesh,
    scratch_types=[
        pltpu.VMEM((_ZR, _F), jnp.float32),
        pltpu.VMEM((_CHUNK // _SUB, _SUB), jnp.int32),
        pltpu.VMEM((_CHUNK // _SUB, _SUB), jnp.int32),
        pltpu.VMEM((_CHUNK,), jnp.float32),
        pltpu.VMEM((_CHUNK, _F), jnp.float32),
        pltpu.VMEM_SHARED((_NPAD, _F), jnp.float32),
        pltpu.SemaphoreType.DMA,
    ],
)


_RB = 1000


def _mm_body(x_ref, w_ref, d0_ref, d1_ref, xw2_ref, dinv_ref):
    deg = d0_ref[...] + d1_ref[...] + 1.0
    dinv = jnp.where(deg > 0, lax.rsqrt(deg), 0.0)
    xw = jnp.dot(x_ref[...], w_ref[...], preferred_element_type=jnp.float32)
    xw2_ref[...] = xw * dinv
    dinv_ref[...] = dinv


_mm_call = pl.pallas_call(
    _mm_body,
    grid=(_N // _RB,),
    in_specs=[
        pl.BlockSpec((_RB, _F), lambda i: (i, 0)),
        pl.BlockSpec((_F, _F), lambda i: (0, 0)),
        pl.BlockSpec((_RB, 1), lambda i: (i, 0)),
        pl.BlockSpec((_RB, 1), lambda i: (i, 0)),
    ],
    out_specs=[
        pl.BlockSpec((_RB, _F), lambda i: (i, 0)),
        pl.BlockSpec((_RB, 1), lambda i: (i, 0)),
    ],
    out_shape=[
        jax.ShapeDtypeStruct((_N, _F), jnp.float32),
        jax.ShapeDtypeStruct((_N, 1), jnp.float32),
    ],
)


def _gru_body(p0_ref, p1_ref, xw2_ref, dinv_ref, gb_ref,
              wz_ref, bz_ref, wh_ref, bh_ref, lw_ref, lb_ref,
              probs_ref, h1_ref):
    h = dinv_ref[...] * (p0_ref[...] + p1_ref[...] + xw2_ref[...]) + gb_ref[...]
    z = jax.nn.sigmoid(
        jnp.dot(h, wz_ref[...], preferred_element_type=jnp.float32) + bz_ref[...])
    ht = jnp.tanh(
        jnp.dot(h, wh_ref[...], preferred_element_type=jnp.float32) + bh_ref[...])
    h1 = (1.0 - z) * ht
    hr = jnp.maximum(h1, 0.0)
    logits = jnp.dot(hr, lw_ref[...], preferred_element_type=jnp.float32) + lb_ref[...]
    m = jnp.max(logits, axis=1, keepdims=True)
    e = jnp.exp(logits - m)
    probs_ref[...] = e / jnp.sum(e, axis=1, keepdims=True)
    h1_ref[...] = h1


_gru_call = pl.pallas_call(
    _gru_body,
    grid=(_N // _RB,),
    in_specs=[
        pl.BlockSpec((_RB, _F), lambda i: (i, 0)),
        pl.BlockSpec((_RB, _F), lambda i: (i, 0)),
        pl.BlockSpec((_RB, _F), lambda i: (i, 0)),
        pl.BlockSpec((_RB, 1), lambda i: (i, 0)),
        pl.BlockSpec((1, _F), lambda i: (0, 0)),
        pl.BlockSpec((_F, _F), lambda i: (0, 0)),
        pl.BlockSpec((1, _F), lambda i: (0, 0)),
        pl.BlockSpec((_F, _F), lambda i: (0, 0)),
        pl.BlockSpec((1, _F), lambda i: (0, 0)),
        pl.BlockSpec((_F, _C), lambda i: (0, 0)),
        pl.BlockSpec((1, _C), lambda i: (0, 0)),
    ],
    out_specs=[
        pl.BlockSpec((_RB, _C), lambda i: (i, 0)),
        pl.BlockSpec((_RB, _F), lambda i: (i, 0)),
    ],
    out_shape=[
        jax.ShapeDtypeStruct((_N, _C), jnp.float32),
        jax.ShapeDtypeStruct((_N, _F), jnp.float32),
    ],
)


def kernel(x, edge_index, edge_weight, gcn_w, gcn_b,
           w_z, b_z, w_r, b_r, w_h, b_h, lin_w, lin_b):
    del w_r, b_r
    pad = _EPAD - _E
    src = jnp.concatenate([edge_index[0], jnp.zeros((pad,), edge_index.dtype)])
    dst = jnp.concatenate([edge_index[1], jnp.zeros((pad,), edge_index.dtype)])
    ewp = jnp.concatenate([edge_weight, jnp.zeros((pad,), edge_weight.dtype)])

    degp = _deg_call(dst, ewp)
    d0 = degp[0, :_N].reshape(_N, 1)
    d1 = degp[1, :_N].reshape(_N, 1)
    xw2, dinv = _mm_call(x, gcn_w, d0, d1)

    aggp = _agg_call(src, dst, ewp, xw2)
    p0 = aggp[0, :_N]
    p1 = aggp[1, :_N]

    wz = (w_z[0, 0] + w_z[1, 0])[:_F]
    wh = (w_h[0, 0] + w_h[1, 0])[:_F]
    probs, h1 = _gru_call(p0, p1, xw2, dinv, gcn_b.reshape(1, _F),
                          wz, b_z.reshape(1, _F), wh, b_h.reshape(1, _F),
                          lin_w, lin_b.reshape(1, _C))
    return probs, h1
